# baseline (speedup 1.0000x reference)
import functools

import jax
import jax.numpy as jnp
from jax import lax
from jax.experimental import pallas as pl
from jax.experimental.pallas import tpu as pltpu
from jax.experimental.pallas import tpu_sc as plsc

N = 10000
F = 128
C = 16
E = 320000
NC = 2
NS = 16
NW = NC * NS
EPW = E // NW
CH1 = 40
NCH1 = EPW // CH1
KB1 = 2
CH2 = CH1
NCH2 = NCH1
KB2 = 5
NIT = 10
RPT = N // NIT

_MESH = plsc.VectorSubcoreMesh(
    core_axis_name="c", subcore_axis_name="s", num_cores=NC, num_subcores=NS)


def _edge_pipeline(tab, src_v, dst_v, rows_v, acc_sh, gsem, ssem, nchunk, kb):
    def fire_g(buf, base):
        for k in range(kb):
            pltpu.async_copy(
                tab.at[src_v.at[base + k]], rows_v.at[buf + k], gsem)

    def drain_g(buf, base):
        for k in range(kb):
            pltpu.make_async_copy(
                tab.at[src_v.at[base + k]], rows_v.at[buf + k], gsem).wait()

    def fire_s(buf, base):
        for k in range(kb):
            pltpu.async_copy(
                rows_v.at[buf + k], acc_sh.at[dst_v.at[base + k]], ssem,
                add=True)

    def drain_s(buf, base):
        for k in range(kb):
            pltpu.make_async_copy(
                rows_v.at[buf + k], acc_sh.at[dst_v.at[base + k]],
                ssem).wait()

    nbatch = nchunk // kb
    npair = nbatch // 2

    fire_g(0, 0)

    def body(i, carry):
        base_a = 2 * i * kb
        drain_g(0, base_a)
        fire_s(0, base_a)

        @pl.when(i > 0)
        def _():
            drain_s(kb, base_a - kb)

        fire_g(kb, base_a + kb)
        drain_g(kb, base_a + kb)
        fire_s(kb, base_a + kb)
        drain_s(0, base_a)

        @pl.when(i < npair - 1)
        def _():
            fire_g(0, base_a + 2 * kb)

        return carry

    lax.fori_loop(0, npair, body, 0)
    drain_s(kb, 2 * npair * kb - kb)

    if nbatch % 2:
        base = (nbatch - 1) * kb
        fire_g(0, base)
        drain_g(0, base)
        fire_s(0, base)
        drain_s(0, base)


@functools.partial(
    pl.kernel,
    out_type=jax.ShapeDtypeStruct((NC, N, C), jnp.float32),
    mesh=_MESH,
    compiler_params=pltpu.CompilerParams(use_tc_tiling_on_sc=False),
    scratch_types=[
        pltpu.VMEM((NCH2, CH2), jnp.int32),
        pltpu.VMEM((CH2, C), jnp.float32),
        pltpu.VMEM_SHARED((N, C), jnp.float32),
    ],
)
def _deg_sc(e_hbm, zeros_hbm, ones_hbm, out_hbm, idx_v, ones_v, acc_sh):
    c = lax.axis_index("c")
    s = lax.axis_index("s")
    w = c * NS + s
    pltpu.sync_copy(e_hbm.at[1, w], idx_v)
    pltpu.sync_copy(ones_hbm, ones_v)

    @pl.when(s < NIT)
    def _():
        pltpu.sync_copy(zeros_hbm.at[pl.ds(s * RPT, RPT)],
                        acc_sh.at[pl.ds(s * RPT, RPT)])
    plsc.subcore_barrier()

    def body(ci, carry):
        pltpu.sync_copy(ones_v, acc_sh.at[idx_v.at[ci]], add=True)
        return carry

    lax.fori_loop(0, NCH2, body, 0)
    plsc.subcore_barrier()

    @pl.when(s < NIT)
    def _():
        pltpu.sync_copy(acc_sh.at[pl.ds(s * RPT, RPT)],
                        out_hbm.at[c, pl.ds(s * RPT, RPT)])


def _make_agg(width, chunk, kb):
    nchunk = EPW // chunk

    @functools.partial(
        pl.kernel,
        out_type=jax.ShapeDtypeStruct((NC, N, width), jnp.float32),
        mesh=_MESH,
        compiler_params=pltpu.CompilerParams(use_tc_tiling_on_sc=False),
        scratch_types=[
            pltpu.VMEM((nchunk, chunk), jnp.int32),
            pltpu.VMEM((nchunk, chunk), jnp.int32),
            pltpu.VMEM((2 * kb, chunk, width), jnp.float32),
            pltpu.VMEM_SHARED((N, width), jnp.float32),
            pltpu.SemaphoreType.DMA,
            pltpu.SemaphoreType.DMA,
        ],
    )
    def _agg(e_hbm, tab_hbm, out_hbm,
             src_v, dst_v, rows_v, acc_sh, gsem, ssem):
        c = lax.axis_index("c")
        s = lax.axis_index("s")
        w = c * NS + s
        pltpu.sync_copy(e_hbm.at[0, w], src_v)
        pltpu.sync_copy(e_hbm.at[1, w], dst_v)
        @pl.when(s < NIT)
        def _():
            pltpu.sync_copy(tab_hbm.at[pl.ds(s * RPT, RPT)],
                            acc_sh.at[pl.ds(s * RPT, RPT)])
        plsc.subcore_barrier()
        _edge_pipeline(tab_hbm, src_v, dst_v, rows_v, acc_sh,
                       gsem, ssem, nchunk, kb)
        plsc.subcore_barrier()

        @pl.when(s < NIT)
        def _():
            pltpu.sync_copy(acc_sh.at[pl.ds(s * RPT, RPT)],
                            out_hbm.at[c, pl.ds(s * RPT, RPT)])

    return _agg


_agg128_sc = _make_agg(F, CH1, KB1)
_agg16_sc = _make_agg(C, CH2, KB2)


_GRID = 10
_BR = N // _GRID


def _tc1_body(degp_ref, x_ref, w1_ref, hp_ref, dinv_ref):
    deg = degp_ref[0] + degp_ref[1] + 1.0
    dinv = lax.rsqrt(deg[:, 0:1])
    h = jnp.dot(x_ref[...], w1_ref[...], preferred_element_type=jnp.float32)
    hp_ref[...] = h * dinv
    dinv_ref[...] = dinv


def _tc2_body(p_ref, hp_ref, dinv_ref, b1_ref, w2_ref, gp_ref):
    ssum = p_ref[0] + p_ref[1] - hp_ref[...]
    h1 = jnp.maximum(dinv_ref[...] * ssum + b1_ref[...], 0.0)
    g = jnp.dot(h1, w2_ref[...], preferred_element_type=jnp.float32)
    gp_ref[...] = g * dinv_ref[...]


def _tc3_body(q_ref, gp_ref, dinv_ref, b2_ref, out_ref):
    t = dinv_ref[...] * (q_ref[0] + q_ref[1] - gp_ref[...]) + b2_ref[...]
    m = jnp.max(t, axis=1, keepdims=True)
    lse = jnp.log(jnp.sum(jnp.exp(t - m), axis=1, keepdims=True)) + m
    out_ref[...] = t - lse


def _row_block(width):
    return pl.BlockSpec((_BR, width), lambda i: (i, 0))


def _part_block(width):
    return pl.BlockSpec((NC, _BR, width), lambda i: (0, i, 0))


def _full_block(r, c):
    return pl.BlockSpec((r, c), lambda i: (0, 0))


def kernel(x, edge_index, W1, b1, W2, b2):
    ei = edge_index.astype(jnp.int32)
    e1 = ei.reshape(2, NW, NCH1, CH1)

    zeros_nc = jnp.zeros((N, C), jnp.float32)
    ones_c = jnp.ones((CH2, C), jnp.float32)

    degp = _deg_sc(e1, zeros_nc, ones_c)

    hp, dinv = pl.pallas_call(
        _tc1_body,
        grid=(_GRID,),
        in_specs=[_part_block(C), _row_block(F), _full_block(F, F)],
        out_specs=[_row_block(F), _row_block(1)],
        out_shape=[
            jax.ShapeDtypeStruct((N, F), jnp.float32),
            jax.ShapeDtypeStruct((N, 1), jnp.float32),
        ],
    )(degp, x, W1)

    p1 = _agg128_sc(e1, hp)

    gp = pl.pallas_call(
        _tc2_body,
        grid=(_GRID,),
        in_specs=[_part_block(F), _row_block(F), _row_block(1),
                  _full_block(1, F), _full_block(F, C)],
        out_specs=_row_block(C),
        out_shape=jax.ShapeDtypeStruct((N, C), jnp.float32),
    )(p1, hp, dinv, b1.reshape(1, F), W2)

    q1 = _agg16_sc(e1, gp)

    out = pl.pallas_call(
        _tc3_body,
        grid=(_GRID,),
        in_specs=[_part_block(C), _row_block(C), _row_block(1),
                  _full_block(1, C)],
        out_specs=_row_block(C),
        out_shape=jax.ShapeDtypeStruct((N, C), jnp.float32),
    )(q1, gp, dinv, b2.reshape(1, C))

    return out


# speedup vs baseline: 1.0781x; 1.0781x over previous
import functools

import jax
import jax.numpy as jnp
from jax import lax
from jax.experimental import pallas as pl
from jax.experimental.pallas import tpu as pltpu
from jax.experimental.pallas import tpu_sc as plsc

N = 10000
F = 128
C = 16
E = 320000
NC = 2
NS = 16
NW = NC * NS
EPW = E // NW
CH1 = 80
NCH1 = EPW // CH1
KB1 = 1
CH2 = CH1
NCH2 = NCH1
KB2 = 5
NIT = 10
RPT = N // NIT

_MESH = plsc.VectorSubcoreMesh(
    core_axis_name="c", subcore_axis_name="s", num_cores=NC, num_subcores=NS)


def _edge_pipeline(tab, src_v, dst_v, rows_v, acc_sh, gsem, ssem, nchunk, kb):
    def fire_g(buf, base):
        for k in range(kb):
            pltpu.async_copy(
                tab.at[src_v.at[base + k]], rows_v.at[buf + k], gsem)

    def drain_g(buf, base):
        for k in range(kb):
            pltpu.make_async_copy(
                tab.at[src_v.at[base + k]], rows_v.at[buf + k], gsem).wait()

    def fire_s(buf, base):
        for k in range(kb):
            pltpu.async_copy(
                rows_v.at[buf + k], acc_sh.at[dst_v.at[base + k]], ssem,
                add=True)

    def drain_s(buf, base):
        for k in range(kb):
            pltpu.make_async_copy(
                rows_v.at[buf + k], acc_sh.at[dst_v.at[base + k]],
                ssem).wait()

    nbatch = nchunk // kb
    npair = nbatch // 2

    fire_g(0, 0)

    def body(i, carry):
        base_a = 2 * i * kb
        drain_g(0, base_a)
        fire_s(0, base_a)

        @pl.when(i > 0)
        def _():
            drain_s(kb, base_a - kb)

        fire_g(kb, base_a + kb)
        drain_g(kb, base_a + kb)
        fire_s(kb, base_a + kb)
        drain_s(0, base_a)

        @pl.when(i < npair - 1)
        def _():
            fire_g(0, base_a + 2 * kb)

        return carry

    lax.fori_loop(0, npair, body, 0)
    drain_s(kb, 2 * npair * kb - kb)

    if nbatch % 2:
        base = (nbatch - 1) * kb
        fire_g(0, base)
        drain_g(0, base)
        fire_s(0, base)
        drain_s(0, base)


@functools.partial(
    pl.kernel,
    out_type=jax.ShapeDtypeStruct((NC, N, C), jnp.float32),
    mesh=_MESH,
    compiler_params=pltpu.CompilerParams(use_tc_tiling_on_sc=False),
    scratch_types=[
        pltpu.VMEM((NCH2, CH2), jnp.int32),
        pltpu.VMEM((CH2, C), jnp.float32),
        pltpu.VMEM_SHARED((N, C), jnp.float32),
    ],
)
def _deg_sc(e_hbm, zeros_hbm, ones_hbm, out_hbm, idx_v, ones_v, acc_sh):
    c = lax.axis_index("c")
    s = lax.axis_index("s")
    w = c * NS + s
    pltpu.sync_copy(e_hbm.at[1, w], idx_v)
    pltpu.sync_copy(ones_hbm, ones_v)

    @pl.when(s < NIT)
    def _():
        pltpu.sync_copy(zeros_hbm.at[pl.ds(s * RPT, RPT)],
                        acc_sh.at[pl.ds(s * RPT, RPT)])
    plsc.subcore_barrier()

    def body(ci, carry):
        pltpu.sync_copy(ones_v, acc_sh.at[idx_v.at[ci]], add=True)
        return carry

    lax.fori_loop(0, NCH2, body, 0)
    plsc.subcore_barrier()

    @pl.when(s < NIT)
    def _():
        pltpu.sync_copy(acc_sh.at[pl.ds(s * RPT, RPT)],
                        out_hbm.at[c, pl.ds(s * RPT, RPT)])


def _make_agg(width, chunk, kb):
    nchunk = EPW // chunk

    @functools.partial(
        pl.kernel,
        out_type=jax.ShapeDtypeStruct((NC, N, width), jnp.float32),
        mesh=_MESH,
        compiler_params=pltpu.CompilerParams(use_tc_tiling_on_sc=False),
        scratch_types=[
            pltpu.VMEM((nchunk, chunk), jnp.int32),
            pltpu.VMEM((nchunk, chunk), jnp.int32),
            pltpu.VMEM((2 * kb, chunk, width), jnp.float32),
            pltpu.VMEM_SHARED((N, width), jnp.float32),
            pltpu.SemaphoreType.DMA,
            pltpu.SemaphoreType.DMA,
        ],
    )
    def _agg(e_hbm, tab_hbm, out_hbm,
             src_v, dst_v, rows_v, acc_sh, gsem, ssem):
        c = lax.axis_index("c")
        s = lax.axis_index("s")
        w = c * NS + s
        pltpu.sync_copy(e_hbm.at[0, w], src_v)
        pltpu.sync_copy(e_hbm.at[1, w], dst_v)
        @pl.when(s < NIT)
        def _():
            pltpu.sync_copy(tab_hbm.at[pl.ds(s * RPT, RPT)],
                            acc_sh.at[pl.ds(s * RPT, RPT)])
        plsc.subcore_barrier()
        _edge_pipeline(tab_hbm, src_v, dst_v, rows_v, acc_sh,
                       gsem, ssem, nchunk, kb)
        plsc.subcore_barrier()

        @pl.when(s < NIT)
        def _():
            pltpu.sync_copy(acc_sh.at[pl.ds(s * RPT, RPT)],
                            out_hbm.at[c, pl.ds(s * RPT, RPT)])

    return _agg


_agg128_sc = _make_agg(F, CH1, KB1)
_agg16_sc = _make_agg(C, CH2, KB2)


_GRID = 10
_BR = N // _GRID


def _tc1_body(degp_ref, x_ref, w1_ref, hp_ref, dinv_ref):
    deg = degp_ref[0] + degp_ref[1] + 1.0
    dinv = lax.rsqrt(deg[:, 0:1])
    h = jnp.dot(x_ref[...], w1_ref[...], preferred_element_type=jnp.float32)
    hp_ref[...] = h * dinv
    dinv_ref[...] = dinv


def _tc2_body(p_ref, hp_ref, dinv_ref, b1_ref, w2_ref, gp_ref):
    ssum = p_ref[0] + p_ref[1] - hp_ref[...]
    h1 = jnp.maximum(dinv_ref[...] * ssum + b1_ref[...], 0.0)
    g = jnp.dot(h1, w2_ref[...], preferred_element_type=jnp.float32)
    gp_ref[...] = g * dinv_ref[...]


def _tc3_body(q_ref, gp_ref, dinv_ref, b2_ref, out_ref):
    t = dinv_ref[...] * (q_ref[0] + q_ref[1] - gp_ref[...]) + b2_ref[...]
    m = jnp.max(t, axis=1, keepdims=True)
    lse = jnp.log(jnp.sum(jnp.exp(t - m), axis=1, keepdims=True)) + m
    out_ref[...] = t - lse


def _row_block(width):
    return pl.BlockSpec((_BR, width), lambda i: (i, 0))


def _part_block(width):
    return pl.BlockSpec((NC, _BR, width), lambda i: (0, i, 0))


def _full_block(r, c):
    return pl.BlockSpec((r, c), lambda i: (0, 0))


def kernel(x, edge_index, W1, b1, W2, b2):
    ei = edge_index.astype(jnp.int32)
    e1 = ei.reshape(2, NW, NCH1, CH1)

    zeros_nc = jnp.zeros((N, C), jnp.float32)
    ones_c = jnp.ones((CH2, C), jnp.float32)

    degp = _deg_sc(e1, zeros_nc, ones_c)

    hp, dinv = pl.pallas_call(
        _tc1_body,
        grid=(_GRID,),
        in_specs=[_part_block(C), _row_block(F), _full_block(F, F)],
        out_specs=[_row_block(F), _row_block(1)],
        out_shape=[
            jax.ShapeDtypeStruct((N, F), jnp.float32),
            jax.ShapeDtypeStruct((N, 1), jnp.float32),
        ],
    )(degp, x, W1)

    p1 = _agg128_sc(e1, hp)

    gp = pl.pallas_call(
        _tc2_body,
        grid=(_GRID,),
        in_specs=[_part_block(F), _row_block(F), _row_block(1),
                  _full_block(1, F), _full_block(F, C)],
        out_specs=_row_block(C),
        out_shape=jax.ShapeDtypeStruct((N, C), jnp.float32),
    )(p1, hp, dinv, b1.reshape(1, F), W2)

    q1 = _agg16_sc(e1, gp)

    out = pl.pallas_call(
        _tc3_body,
        grid=(_GRID,),
        in_specs=[_part_block(C), _row_block(C), _row_block(1),
                  _full_block(1, C)],
        out_specs=_row_block(C),
        out_shape=jax.ShapeDtypeStruct((N, C), jnp.float32),
    )(q1, gp, dinv, b2.reshape(1, C))

    return out
